# baseline (speedup 1.0000x reference)
import jax
import jax.numpy as jnp
from jax import lax
from jax.experimental import pallas as pl
from jax.experimental.pallas import tpu as pltpu
from jax.experimental.pallas import tpu_sc as plsc

_PF = plsc.PackFormat.INTERLEAVED

_B, _C, _H, _W = 4, 16, 512, 1024
_F, _GRID = 80, 66
_K = _GRID * _GRID
_N = _F * _K
_CH = _B * _C
_NC, _NS, _L = 2, 16, 16
_NW = _NC * _NS
_CHUNK = 96
_NCHUNK = _N // _CHUNK
_PAIRS = _NCHUNK // 2
_PAIRS_LO = _PAIRS // _NW
_PAIRS_EXTRA = _PAIRS % _NW
_MAXPAIRS = _PAIRS_LO + 1
_MAXSPAN = 2 * _MAXPAIRS * _CHUNK
_WF = _MAXSPAN // _K + 2
_WSIZE = _WF * 2 * _K


def _sc_body(table, smf, out, sma, idx, wgt, rows, outt, gsem, osem):
    wid = lax.axis_index("s") * _NC + lax.axis_index("c")
    npairs = _PAIRS_LO + (wid < _PAIRS_EXTRA).astype(jnp.int32)
    start = 2 * (wid * _PAIRS_LO + jnp.minimum(wid, _PAIRS_EXTRA))
    base_s = start * _CHUNK

    w0 = (base_s // _K) * (2 * _K)
    w0 = jnp.minimum(w0, 2 * _N - _WSIZE)
    pltpu.sync_copy(smf.at[pl.ds(w0, _WSIZE)], sma)

    iota = lax.iota(jnp.int32, _L)
    rowe = [cv * 2 * _L + 2 * iota for cv in range(_CH // (2 * _L))]
    rowo = [cv * 2 * _L + 2 * iota + 1 for cv in range(_CH // (2 * _L))]

    def fire(g, p):
        for j in range(_CHUNK // _L):
            d = pl.ds(j * _L, _L)
            sv = base_s + g * _CHUNK + j * _L + iota
            pos = sv + (sv // _K) * _K - w0
            sxj = plsc.load_gather(sma, [pos])
            syj = plsc.load_gather(sma, [pos + _K])
            x0 = sxj.astype(jnp.int32)
            y0 = syj.astype(jnp.int32)
            wgt[p][0][d] = sxj - x0.astype(jnp.float32)
            wgt[p][1][d] = syj - y0.astype(jnp.float32)
            x0 = jnp.remainder(x0, _W)
            x1 = jnp.remainder(x0 + 1, _W)
            y0 = jnp.clip(y0, 0, _H - 1)
            y1 = jnp.minimum(y0 + 1, _H - 1)
            r0 = y0 * _W
            r1 = y1 * _W
            idx[p][0][d] = r0 + x0
            idx[p][1][d] = r0 + x1
            idx[p][2][d] = r1 + x0
            idx[p][3][d] = r1 + x1
        for q in range(4):
            pltpu.async_copy(table.at[idx[p][q]], rows[p][q], gsem[p])

    fire(0, 0)

    def outer(gg, carry):
        @pl.when(gg < npairs)
        def _():
            for b in range(2):
                g = gg * 2 + b

                if b == 0:
                    fire(g + 1, 1 - b)
                else:
                    @pl.when(gg + 1 < npairs)
                    def _():
                        fire(g + 1, 1 - b)

                for q in range(4):
                    pltpu.make_async_copy(
                        table.at[pl.ds(0, _CHUNK)], rows[b][q], gsem[b]).wait()

                @pl.when(gg > 0)
                def _():
                    pltpu.make_async_copy(
                        outt[b].at[:, pl.ds(0, _CHUNK)],
                        out.at[:, pl.ds(0, _CHUNK)], osem[b]).wait()

                r00, r01, r10, r11 = rows[b]

                def grp_body(jg, c):
                    s = pl.ds(jg * _L, _L)
                    wxg = wgt[b][0][s]
                    wyg = wgt[b][1][s]
                    w11g = wxg * wyg
                    w10g = wyg - w11g
                    w01g = wxg - w11g
                    w00g = 1.0 - wxg - wyg + w11g
                    for l in range(_L):
                        i = jg * _L + l
                        coli = jnp.full((_L,), i, jnp.int32)
                        w00, w01 = w00g[l], w01g[l]
                        w10, w11 = w10g[l], w11g[l]
                        for cv in range(_CH // (2 * _L)):
                            t = pl.ds(cv * 2 * _L, 2 * _L)
                            v00e, v00o = plsc.unpack(r00[i, t], format=_PF)
                            v01e, v01o = plsc.unpack(r01[i, t], format=_PF)
                            v10e, v10o = plsc.unpack(r10[i, t], format=_PF)
                            v11e, v11o = plsc.unpack(r11[i, t], format=_PF)
                            acce = (v00e * w00 + v01e * w01
                                    + v10e * w10 + v11e * w11)
                            acco = (v00o * w00 + v01o * w01
                                    + v10o * w10 + v11o * w11)
                            plsc.store_scatter(outt[b], [rowe[cv], coli], acce)
                            plsc.store_scatter(outt[b], [rowo[cv], coli], acco)
                    return c

                lax.fori_loop(0, _CHUNK // _L, grp_body, 0, unroll=2)
                off = (start + g) * _CHUNK
                pltpu.async_copy(outt[b].at[:, pl.ds(0, _CHUNK)],
                                 out.at[:, pl.ds(off, _CHUNK)], osem[b])
        return carry

    lax.fori_loop(0, _MAXPAIRS, outer, 0)
    for b in range(2):
        pltpu.make_async_copy(outt[b].at[:, pl.ds(0, _CHUNK)],
                              out.at[:, pl.ds(0, _CHUNK)], osem[b]).wait()


def kernel(x, sample_map):
    table = x.reshape(_CH, _H * _W).T.astype(jnp.bfloat16)
    smf = jnp.transpose(sample_map, (0, 2, 1)).reshape(2 * _N)
    mesh = plsc.VectorSubcoreMesh(core_axis_name="c", subcore_axis_name="s")
    out_t = pl.kernel(
        _sc_body,
        out_type=jax.ShapeDtypeStruct((_CH, _N), jnp.float32),
        mesh=mesh,
        compiler_params=pltpu.CompilerParams(use_tc_tiling_on_sc=False,
                                             needs_layout_passes=False),
        scratch_types=[
            pltpu.VMEM((_WSIZE,), jnp.float32),
            [[pltpu.VMEM((_CHUNK,), jnp.int32) for _ in range(4)]
             for _ in range(2)],
            [[pltpu.VMEM((_CHUNK,), jnp.float32) for _ in range(2)]
             for _ in range(2)],
            [[pltpu.VMEM((_CHUNK, _CH), jnp.bfloat16) for _ in range(4)]
             for _ in range(2)],
            [pltpu.VMEM((_CH, _CHUNK + 1), jnp.float32)
             for _ in range(2)],
            [pltpu.SemaphoreType.DMA for _ in range(2)],
            [pltpu.SemaphoreType.DMA for _ in range(2)],
        ],
    )(table, smf)
    return out_t.reshape(_B, _C, _F, _GRID, _GRID)


# speedup vs baseline: 1.1152x; 1.1152x over previous
import jax
import jax.numpy as jnp
from jax import lax
from jax.experimental import pallas as pl
from jax.experimental.pallas import tpu as pltpu
from jax.experimental.pallas import tpu_sc as plsc

_PF = plsc.PackFormat.INTERLEAVED

_B, _C, _H, _W = 4, 16, 512, 1024
_F, _GRID = 80, 66
_K = _GRID * _GRID
_N = _F * _K
_CH = _B * _C
_NC, _NS, _L = 2, 16, 16
_NW = _NC * _NS
_CHUNK = 96
_NCHUNK = _N // _CHUNK
_PAIRS = _NCHUNK // 2
_PAIRS_LO = _PAIRS // _NW
_PAIRS_EXTRA = _PAIRS % _NW
_MAXPAIRS = _PAIRS_LO + 1
_MAXSPAN = 2 * _MAXPAIRS * _CHUNK
_WF = _MAXSPAN // _K + 2
_WSIZE = _WF * 2 * _K


def _sc_body(table, smf, out, sma, idx, wgt, rows, outt, gsem, osem):
    wid = lax.axis_index("s") * _NC + lax.axis_index("c")
    npairs = _PAIRS_LO + (wid < _PAIRS_EXTRA).astype(jnp.int32)
    start = 2 * (wid * _PAIRS_LO + jnp.minimum(wid, _PAIRS_EXTRA))
    base_s = start * _CHUNK

    w0 = (base_s // _K) * (2 * _K)
    w0 = jnp.minimum(w0, 2 * _N - _WSIZE)
    pltpu.sync_copy(smf.at[pl.ds(w0, _WSIZE)], sma)

    iota = lax.iota(jnp.int32, _L)
    rowe = [cv * 2 * _L + 2 * iota for cv in range(_CH // (2 * _L))]
    rowo = [cv * 2 * _L + 2 * iota + 1 for cv in range(_CH // (2 * _L))]

    def fire(g, p):
        for j in range(_CHUNK // _L):
            d = pl.ds(j * _L, _L)
            sv = base_s + g * _CHUNK + j * _L + iota
            pos = sv + (sv // _K) * _K - w0
            sxj = plsc.load_gather(sma, [pos])
            syj = plsc.load_gather(sma, [pos + _K])
            x0 = sxj.astype(jnp.int32)
            y0 = syj.astype(jnp.int32)
            wgt[p][0][d] = sxj - x0.astype(jnp.float32)
            wgt[p][1][d] = syj - y0.astype(jnp.float32)
            x0 = jnp.remainder(x0, _W)
            x1 = jnp.remainder(x0 + 1, _W)
            y0 = jnp.clip(y0, 0, _H - 1)
            y1 = jnp.minimum(y0 + 1, _H - 1)
            r0 = y0 * _W
            r1 = y1 * _W
            idx[p][0][d] = r0 + x0
            idx[p][1][d] = r0 + x1
            idx[p][2][d] = r1 + x0
            idx[p][3][d] = r1 + x1
        for q in range(4):
            pltpu.async_copy(table.at[idx[p][q]], rows[p][q], gsem[p])

    fire(0, 0)

    def outer(gg, carry):
        @pl.when(gg < npairs)
        def _():
            for b in range(2):
                g = gg * 2 + b

                if b == 0:
                    fire(g + 1, 1 - b)
                else:
                    @pl.when(gg + 1 < npairs)
                    def _():
                        fire(g + 1, 1 - b)

                for q in range(4):
                    pltpu.make_async_copy(
                        table.at[pl.ds(0, _CHUNK)], rows[b][q], gsem[b]).wait()

                @pl.when(gg > 0)
                def _():
                    pltpu.make_async_copy(
                        outt[b].at[:, pl.ds(0, _CHUNK)],
                        out.at[:, pl.ds(0, _CHUNK)], osem[b]).wait()

                r00, r01, r10, r11 = rows[b]

                def grp_body(jg, c):
                    s = pl.ds(jg * _L, _L)
                    wxg = wgt[b][0][s]
                    wyg = wgt[b][1][s]
                    w11g = wxg * wyg
                    w10g = wyg - w11g
                    w01g = wxg - w11g
                    w00g = 1.0 - wxg - wyg + w11g
                    for l in range(_L):
                        i = jg * _L + l
                        coli = jnp.full((_L,), i, jnp.int32)
                        w00, w01 = w00g[l], w01g[l]
                        w10, w11 = w10g[l], w11g[l]
                        for cv in range(_CH // (2 * _L)):
                            t = pl.ds(cv * 2 * _L, 2 * _L)
                            v00e, v00o = plsc.unpack(r00[i, t], format=_PF)
                            v01e, v01o = plsc.unpack(r01[i, t], format=_PF)
                            v10e, v10o = plsc.unpack(r10[i, t], format=_PF)
                            v11e, v11o = plsc.unpack(r11[i, t], format=_PF)
                            acce = (v00e * w00 + v01e * w01
                                    + v10e * w10 + v11e * w11)
                            acco = (v00o * w00 + v01o * w01
                                    + v10o * w10 + v11o * w11)
                            plsc.store_scatter(outt[b], [rowe[cv], coli], acce)
                            plsc.store_scatter(outt[b], [rowo[cv], coli], acco)
                    return c

                lax.fori_loop(0, _CHUNK // _L, grp_body, 0)
                off = (start + g) * _CHUNK
                pltpu.async_copy(outt[b].at[:, pl.ds(0, _CHUNK)],
                                 out.at[:, pl.ds(off, _CHUNK)], osem[b])
        return carry

    lax.fori_loop(0, _MAXPAIRS, outer, 0)
    for b in range(2):
        pltpu.make_async_copy(outt[b].at[:, pl.ds(0, _CHUNK)],
                              out.at[:, pl.ds(0, _CHUNK)], osem[b]).wait()


def kernel(x, sample_map):
    table = x.reshape(_CH, _H * _W).T.astype(jnp.bfloat16)
    smf = jnp.transpose(sample_map, (0, 2, 1)).reshape(2 * _N)
    mesh = plsc.VectorSubcoreMesh(core_axis_name="c", subcore_axis_name="s")
    out_t = pl.kernel(
        _sc_body,
        out_type=jax.ShapeDtypeStruct((_CH, _N), jnp.float32),
        mesh=mesh,
        compiler_params=pltpu.CompilerParams(use_tc_tiling_on_sc=False,
                                             needs_layout_passes=False),
        scratch_types=[
            pltpu.VMEM((_WSIZE,), jnp.float32),
            [[pltpu.VMEM((_CHUNK,), jnp.int32) for _ in range(4)]
             for _ in range(2)],
            [[pltpu.VMEM((_CHUNK,), jnp.float32) for _ in range(2)]
             for _ in range(2)],
            [[pltpu.VMEM((_CHUNK, _CH), jnp.bfloat16) for _ in range(4)]
             for _ in range(2)],
            [pltpu.VMEM((_CH, _CHUNK + 1), jnp.float32)
             for _ in range(2)],
            [pltpu.SemaphoreType.DMA for _ in range(2)],
            [pltpu.SemaphoreType.DMA for _ in range(2)],
        ],
    )(table, smf)
    return out_t.reshape(_B, _C, _F, _GRID, _GRID)
